# Initial kernel scaffold; baseline (speedup 1.0000x reference)
import functools

import jax
import jax.numpy as jnp
from jax import lax
from jax.experimental import pallas as pl
from jax.experimental.pallas import tpu as pltpu
from jax.experimental.pallas import tpu_sc as plsc

N = 10000
D = 128
E = 320000

NC = 2
NS = 16
NW = NC * NS
CH = 128
NCH = -(-E // (NW * CH))
EPAD = NW * CH * NCH
ACC_ROWS = ((N + 1 + NS - 1) // NS) * NS
RPT = ACC_ROWS // NS


def _mesh():
    return plsc.VectorSubcoreMesh(core_axis_name="c", subcore_axis_name="s")


@functools.partial(
    pl.kernel,
    out_type=jax.ShapeDtypeStruct((NC, ACC_ROWS, 16), jnp.float32),
    mesh=_mesh(),
    scratch_types=[
        pltpu.VMEM((CH,), jnp.int32),
        pltpu.VMEM((CH, 16), jnp.float32),
        pltpu.VMEM_SHARED((ACC_ROWS, 16), jnp.float32),
    ],
)
def _deg_kernel(dst_hbm, zeros_hbm, ones_hbm, out_hbm, idx_v, ones_v, acc):
    cid = lax.axis_index("c")
    sid = lax.axis_index("s")
    wid = cid * NS + sid
    r0 = sid * RPT
    pltpu.sync_copy(zeros_hbm.at[pl.ds(r0, RPT)], acc.at[pl.ds(r0, RPT)])
    pltpu.sync_copy(ones_hbm, ones_v)
    plsc.subcore_barrier()
    base0 = wid * (NCH * CH)

    @pl.loop(0, NCH)
    def _(i):
        pltpu.sync_copy(dst_hbm.at[pl.ds(base0 + i * CH, CH)], idx_v)
        pltpu.sync_copy(ones_v, acc.at[idx_v], add=True)

    plsc.subcore_barrier()
    pltpu.sync_copy(acc.at[pl.ds(r0, RPT)], out_hbm.at[cid, pl.ds(r0, RPT)])


@functools.partial(
    pl.kernel,
    out_type=jax.ShapeDtypeStruct((NC, ACC_ROWS, D), jnp.float32),
    mesh=_mesh(),
    scratch_types=[
        pltpu.VMEM((CH,), jnp.int32),
        pltpu.VMEM((CH,), jnp.int32),
        pltpu.VMEM((CH, D), jnp.float32),
        pltpu.VMEM_SHARED((ACC_ROWS, D), jnp.float32),
    ],
)
def _agg_kernel(src_hbm, dst_hbm, h_hbm, zeros_hbm, out_hbm,
                sidx_v, didx_v, rows_v, acc):
    cid = lax.axis_index("c")
    sid = lax.axis_index("s")
    wid = cid * NS + sid
    r0 = sid * RPT
    pltpu.sync_copy(zeros_hbm.at[pl.ds(r0, RPT)], acc.at[pl.ds(r0, RPT)])
    plsc.subcore_barrier()
    base0 = wid * (NCH * CH)

    @pl.loop(0, NCH)
    def _(i):
        pltpu.sync_copy(src_hbm.at[pl.ds(base0 + i * CH, CH)], sidx_v)
        pltpu.sync_copy(dst_hbm.at[pl.ds(base0 + i * CH, CH)], didx_v)
        pltpu.sync_copy(h_hbm.at[sidx_v], rows_v)
        pltpu.sync_copy(rows_v, acc.at[didx_v], add=True)

    plsc.subcore_barrier()
    pltpu.sync_copy(acc.at[pl.ds(r0, RPT)], out_hbm.at[cid, pl.ds(r0, RPT)])


def _h_body(x_ref, w_ref, b_ref, deg_ref, h_ref):
    deg = 1.0 + deg_ref[0, :N, 0:1] + deg_ref[1, :N, 0:1]
    dinv = lax.rsqrt(deg)
    h = jnp.dot(x_ref[...], w_ref[...], preferred_element_type=jnp.float32)
    h_ref[...] = (h + b_ref[...]) * dinv


_h_call = pl.pallas_call(
    _h_body, out_shape=jax.ShapeDtypeStruct((N, D), jnp.float32))


def _out_body(agg_ref, h_ref, deg_ref, x_ref, g_ref, bt_ref, o_ref):
    deg = 1.0 + deg_ref[0, :N, 0:1] + deg_ref[1, :N, 0:1]
    dinv = lax.rsqrt(deg)
    pre = (agg_ref[0, :N, :] + agg_ref[1, :N, :] + h_ref[...]) * dinv
    mean = jnp.mean(pre, axis=0, keepdims=True)
    cen = pre - mean
    var = jnp.mean(cen * cen, axis=0, keepdims=True)
    y = cen * lax.rsqrt(var + 1e-5) * g_ref[...] + bt_ref[...]
    o_ref[...] = jnp.maximum(y, 0.0) + x_ref[...]


_out_call = pl.pallas_call(
    _out_body, out_shape=jax.ShapeDtypeStruct((N, D), jnp.float32))


def kernel(x, edge_index, W, b, gamma, beta):
    src = edge_index[0].astype(jnp.int32)
    dst = edge_index[1].astype(jnp.int32)
    npad = EPAD - E
    src_p = jnp.concatenate([src, jnp.zeros((npad,), jnp.int32)])
    dst_p = jnp.concatenate([dst, jnp.full((npad,), N, jnp.int32)])
    zeros16 = jnp.zeros((ACC_ROWS, 16), jnp.float32)
    ones16 = jnp.ones((CH, 16), jnp.float32)
    zerosD = jnp.zeros((ACC_ROWS, D), jnp.float32)

    degacc = _deg_kernel(dst_p, zeros16, ones16)
    h = _h_call(x, W, b.reshape(1, D), degacc)
    agg = _agg_kernel(src_p, dst_p, h, zerosD)
    return _out_call(agg, h, degacc, x,
                     gamma.reshape(1, D), beta.reshape(1, D))


# speedup vs baseline: 13.6792x; 13.6792x over previous
import functools

import jax
import jax.numpy as jnp
from jax import lax
from jax.experimental import pallas as pl
from jax.experimental.pallas import tpu as pltpu
from jax.experimental.pallas import tpu_sc as plsc

N = 10000
D = 128
E = 320000

NC = 2
NS = 16
NW = NC * NS
CH = 128
NCH = -(-E // (NW * CH))
EPAD = NW * CH * NCH
ACC_ROWS = ((N + 1 + NS * 8 - 1) // (NS * 8)) * (NS * 8)
RPT = ACC_ROWS // NS


def _mesh():
    return plsc.VectorSubcoreMesh(core_axis_name="c", subcore_axis_name="s")


DEG_LEN = ACC_ROWS * 16


@functools.partial(
    pl.kernel,
    out_type=jax.ShapeDtypeStruct((NC * DEG_LEN,), jnp.float32),
    mesh=_mesh(),
    scratch_types=[
        pltpu.VMEM((CH,), jnp.int32),
        pltpu.VMEM((CH,), jnp.float32),
        pltpu.VMEM_SHARED((DEG_LEN,), jnp.float32),
    ],
)
def _deg_kernel(dst16_hbm, zeros_hbm, ones_hbm, out_hbm, idx_v, ones_v, acc):
    cid = lax.axis_index("c")
    sid = lax.axis_index("s")
    wid = cid * NS + sid
    r0 = sid * (RPT * 16)
    pltpu.sync_copy(zeros_hbm.at[pl.ds(r0, RPT * 16)], acc.at[pl.ds(r0, RPT * 16)])
    pltpu.sync_copy(ones_hbm, ones_v)
    plsc.subcore_barrier()
    base0 = wid * (NCH * CH)

    @pl.loop(0, NCH)
    def _(i):
        pltpu.sync_copy(dst16_hbm.at[pl.ds(base0 + i * CH, CH)], idx_v)
        pltpu.sync_copy(ones_v, acc.at[idx_v], add=True)

    plsc.subcore_barrier()
    pltpu.sync_copy(acc.at[pl.ds(r0, RPT * 16)],
                    out_hbm.at[pl.ds(cid * DEG_LEN + r0, RPT * 16)])


@functools.partial(
    pl.kernel,
    out_type=jax.ShapeDtypeStruct((NC, ACC_ROWS, D), jnp.float32),
    mesh=_mesh(),
    scratch_types=[
        pltpu.VMEM((CH,), jnp.int32),
        pltpu.VMEM((CH,), jnp.int32),
        pltpu.VMEM((CH, D), jnp.float32),
        pltpu.VMEM_SHARED((ACC_ROWS, D), jnp.float32),
    ],
)
def _agg_kernel(src_hbm, dst_hbm, h_hbm, zeros_hbm, out_hbm,
                sidx_v, didx_v, rows_v, acc):
    cid = lax.axis_index("c")
    sid = lax.axis_index("s")
    wid = cid * NS + sid
    r0 = sid * RPT
    pltpu.sync_copy(zeros_hbm.at[pl.ds(r0, RPT)], acc.at[pl.ds(r0, RPT)])
    plsc.subcore_barrier()
    base0 = wid * (NCH * CH)

    @pl.loop(0, NCH)
    def _(i):
        pltpu.sync_copy(src_hbm.at[pl.ds(base0 + i * CH, CH)], sidx_v)
        pltpu.sync_copy(dst_hbm.at[pl.ds(base0 + i * CH, CH)], didx_v)
        pltpu.sync_copy(h_hbm.at[sidx_v], rows_v)
        pltpu.sync_copy(rows_v, acc.at[didx_v], add=True)

    plsc.subcore_barrier()
    pltpu.sync_copy(acc.at[pl.ds(r0, RPT)], out_hbm.at[cid, pl.ds(r0, RPT)])


def _h_body(x_ref, w_ref, b_ref, deg_ref, h_ref):
    deg = 1.0 + deg_ref[0, :N, 0:1] + deg_ref[1, :N, 0:1]
    dinv = lax.rsqrt(deg)
    h = jnp.dot(x_ref[...], w_ref[...], preferred_element_type=jnp.float32)
    h_ref[...] = (h + b_ref[...]) * dinv


_h_call = pl.pallas_call(
    _h_body, out_shape=jax.ShapeDtypeStruct((N, D), jnp.float32))


def _out_body(agg_ref, h_ref, deg_ref, x_ref, g_ref, bt_ref, o_ref):
    deg = 1.0 + deg_ref[0, :N, 0:1] + deg_ref[1, :N, 0:1]
    dinv = lax.rsqrt(deg)
    pre = (agg_ref[0, :N, :] + agg_ref[1, :N, :] + h_ref[...]) * dinv
    mean = jnp.mean(pre, axis=0, keepdims=True)
    cen = pre - mean
    var = jnp.mean(cen * cen, axis=0, keepdims=True)
    y = cen * lax.rsqrt(var + 1e-5) * g_ref[...] + bt_ref[...]
    o_ref[...] = jnp.maximum(y, 0.0) + x_ref[...]


_out_call = pl.pallas_call(
    _out_body, out_shape=jax.ShapeDtypeStruct((N, D), jnp.float32))


def kernel(x, edge_index, W, b, gamma, beta):
    src = edge_index[0].astype(jnp.int32)
    dst = edge_index[1].astype(jnp.int32)
    npad = EPAD - E
    src_p = jnp.concatenate([src, jnp.zeros((npad,), jnp.int32)])
    dst_p = jnp.concatenate([dst, jnp.full((npad,), N, jnp.int32)])
    dst16_p = dst_p * 16
    zeros16 = jnp.zeros((DEG_LEN,), jnp.float32)
    ones16 = jnp.ones((CH,), jnp.float32)
    zerosD = jnp.zeros((ACC_ROWS, D), jnp.float32)

    degacc = _deg_kernel(dst16_p, zeros16, ones16).reshape(NC, ACC_ROWS, 16)
    h = _h_call(x, W, b.reshape(1, D), degacc)
    agg = _agg_kernel(src_p, dst_p, h, zerosD)
    return _out_call(agg, h, degacc, x,
                     gamma.reshape(1, D), beta.reshape(1, D))
